# baseline (speedup 1.0000x reference)
import functools

import jax
import jax.numpy as jnp
import numpy as np
from jax import lax
from jax.experimental import pallas as pl
from jax.experimental.pallas import tpu as pltpu
from jax.experimental.pallas import tpu_sc as plsc

_B = 4
_N = 8192
_C = 64
_NPT = 1024
_NS = 32
_R2 = np.float32(0.2 * 0.2)


def _redmax(a):
    for w in (512, 256, 128):
        a = jnp.maximum(a[:, :, :w], a[:, :, w:])
    a32 = jnp.max(a.reshape(_B * 8, 128), axis=1, keepdims=True)
    return jnp.max(a32.reshape(_B, 8, 1), axis=1, keepdims=True)


def _redmin(a):
    for w in (512, 256, 128):
        a = jnp.minimum(a[:, :, :w], a[:, :, w:])
    a32 = jnp.min(a.reshape(_B * 8, 128), axis=1, keepdims=True)
    return jnp.min(a32.reshape(_B, 8, 1), axis=1, keepdims=True)


def _fps_body(xs_ref, ys_ref, zs_ref, cent_ref, dist_ref):
    flat3 = (lax.broadcasted_iota(jnp.int32, (_B, 8, 1024), 1) * 1024
             + lax.broadcasted_iota(jnp.int32, (_B, 8, 1024), 2))
    lane16 = lax.broadcasted_iota(jnp.int32, (1, 16), 1)
    dist_ref[...] = jnp.full((_B, 8, 1024), 1e10, jnp.float32)

    def step(i, carry):
        cx4, cy4, cz4 = carry
        crow = jnp.zeros((1, 16), jnp.float32)
        for b in range(_B):
            crow = jnp.where(lane16 == (4 * b + 0), cx4[b, 0:1, :], crow)
            crow = jnp.where(lane16 == (4 * b + 1), cy4[b, 0:1, :], crow)
            crow = jnp.where(lane16 == (4 * b + 2), cz4[b, 0:1, :], crow)
        cent_ref[pl.ds(i, 1), :] = crow
        xv = xs_ref[...]
        yv = ys_ref[...]
        zv = zs_ref[...]
        dx = xv - cx4
        dy = yv - cy4
        dz = zv - cz4
        d = (dx * dx + dy * dy) + dz * dz
        d4n = jnp.minimum(dist_ref[...], d)
        dist_ref[...] = d4n
        m4 = _redmax(d4n)
        cand = jnp.where(d4n == m4, flat3, jnp.int32(_N))
        fmin4 = _redmin(cand)
        maskf = flat3 == fmin4
        ncx = _redmax(jnp.where(maskf, xv, -1.0))
        ncy = _redmax(jnp.where(maskf, yv, -1.0))
        ncz = _redmax(jnp.where(maskf, zv, -1.0))
        return (ncx, ncy, ncz)

    c0 = tuple(r[:, 0:1, 0:1] for r in (xs_ref, ys_ref, zs_ref))
    lax.fori_loop(0, _NPT, step, c0)


@functools.partial(jax.jit)
def _run_fps(xyz):
    planes = jnp.transpose(xyz, (0, 2, 1)).reshape(_B, 3, 8, 1024)
    cent = pl.pallas_call(
        _fps_body,
        out_shape=jax.ShapeDtypeStruct((_NPT, 16), jnp.float32),
        scratch_shapes=[pltpu.VMEM((_B, 8, 1024), jnp.float32)],
    )(planes[:, 0], planes[:, 1], planes[:, 2])
    return cent


_NW = 32


def _bq_body(nx_hbm, xyzt_hbm, feat_hbm, out_hbm,
             cx_v, cy_v, cz_v, xs_v, ys_v, zs_v, idx_v, gbuf_v, sem):
    cid = lax.axis_index("c")
    sid = lax.axis_index("s")
    wid = sid * 2 + cid
    b = wid // 8
    part = wid % 8
    lane = lax.iota(jnp.int32, 16)

    pltpu.sync_copy(nx_hbm.at[pl.ds((b * 3 + 0) * _NPT + part * 128, 128)], cx_v)
    pltpu.sync_copy(nx_hbm.at[pl.ds((b * 3 + 1) * _NPT + part * 128, 128)], cy_v)
    pltpu.sync_copy(nx_hbm.at[pl.ds((b * 3 + 2) * _NPT + part * 128, 128)], cz_v)
    pltpu.sync_copy(xyzt_hbm.at[pl.ds((b * 3 + 0) * _N, _N)], xs_v)
    pltpu.sync_copy(xyzt_hbm.at[pl.ds((b * 3 + 1) * _N, _N)], ys_v)
    pltpu.sync_copy(xyzt_hbm.at[pl.ds((b * 3 + 2) * _N, _N)], zs_v)

    rowbase = b * _N
    zeros16 = jnp.zeros((16,), jnp.int32)

    def one_centroid(si, _):
        g = si // 16
        j = si - g * 16
        js = jnp.full((16,), j, jnp.int32)
        cxs = cx_v[pl.ds(g * 16, 16)].at[js].get(mode="promise_in_bounds")
        cys = cy_v[pl.ds(g * 16, 16)].at[js].get(mode="promise_in_bounds")
        czs = cz_v[pl.ds(g * 16, 16)].at[js].get(mode="promise_in_bounds")

        def chunk(k, cnt):
            base = k * 32
            for h in (0, 16):
                dx = xs_v[pl.ds(base + h, 16)] - cxs
                dy = ys_v[pl.ds(base + h, 16)] - cys
                dz = zs_v[pl.ds(base + h, 16)] - czs
                d = (dx * dx + dy * dy) + dz * dz
                inr = d <= _R2
                npts = plsc.all_reduce_population_count(inr)
                n = npts[0]
                ivec = lane + jnp.full((16,), base + h + rowbase, jnp.int32)
                off = si * _NS + jnp.minimum(cnt, _NS)

                @pl.when(n > 0)
                def _():
                    plsc.store_compressed(idx_v.at[pl.ds(off, 16)], ivec,
                                          mask=inr)

                cnt = cnt + n
            return cnt

        cnt = jnp.int32(0)
        k_cur = jnp.int32(0)
        for rl in (4, 4, 4, 4, 8, 8, 16, 32, 64, 112):
            k_end = jnp.where(cnt >= _NS, k_cur, k_cur + rl)
            cnt = pl.loop(k_cur, k_end, init_carry=cnt)(chunk)
            k_cur = k_end

        cnt32 = jnp.minimum(cnt, _NS)
        v0 = idx_v[pl.ds(si * _NS, 16)]
        first_raw = v0.at[zeros16].get(mode="promise_in_bounds")
        firsts = jnp.where(cnt32 > 0, first_raw, jnp.full((16,), rowbase, jnp.int32))
        for h in range(2):
            cur = idx_v[pl.ds(si * _NS + h * 16, 16)]
            gl = lane + h * 16
            idx_v[pl.ds(si * _NS + h * 16, 16)] = jnp.where(gl < cnt32, cur, firsts)
        return 0

    lax.fori_loop(0, 128, one_centroid, 0)

    out_base = (b * _NPT + part * 128) * _NS

    def gather_chunk(ci, _):
        pltpu.async_copy(feat_hbm.at[idx_v.at[pl.ds(ci * 128, 128)]], gbuf_v, sem).wait()
        pltpu.sync_copy(gbuf_v, out_hbm.at[pl.ds(out_base + ci * 128, 128)])
        return 0

    lax.fori_loop(0, _NS, gather_chunk, 0)


def _make_bq():
    mesh = plsc.VectorSubcoreMesh(core_axis_name="c", subcore_axis_name="s")
    return functools.partial(
        pl.kernel,
        mesh=mesh,
        compiler_params=pltpu.CompilerParams(needs_layout_passes=False,
                                             use_tc_tiling_on_sc=False),
        out_type=jax.ShapeDtypeStruct((_B * _NPT * _NS, _C), jnp.float32),
        scratch_types=[
            pltpu.VMEM((128,), jnp.float32),
            pltpu.VMEM((128,), jnp.float32),
            pltpu.VMEM((128,), jnp.float32),
            pltpu.VMEM((_N,), jnp.float32),
            pltpu.VMEM((_N,), jnp.float32),
            pltpu.VMEM((_N,), jnp.float32),
            pltpu.VMEM((128 * _NS + 16,), jnp.int32),
            pltpu.VMEM((128, _C), jnp.float32),
            pltpu.SemaphoreType.DMA,
        ],
    )(_bq_body)


_BC = 64


def _mlp_body(g_ref, w1_ref, b1_ref, w2_ref, b2_ref, w3_ref, b3_ref, out_ref):
    x = g_ref[...].reshape(_BC * _NS, _C)
    h = jnp.dot(x, w1_ref[...], preferred_element_type=jnp.float32) + b1_ref[...]
    h = jnp.maximum(h, 0.0)
    h = jnp.dot(h, w2_ref[...], preferred_element_type=jnp.float32) + b2_ref[...]
    h = jnp.maximum(h, 0.0)
    h = jnp.dot(h, w3_ref[...], preferred_element_type=jnp.float32) + b3_ref[...]
    out_ref[...] = jnp.max(h.reshape(_BC, _NS, 256), axis=1)


def _run_mlp(grouped, W1, b1, W2, b2, W3, b3):
    g3 = grouped.reshape(_B * _NPT, _NS, _C)
    out = pl.pallas_call(
        _mlp_body,
        grid=(_B * _NPT // _BC,),
        in_specs=[
            pl.BlockSpec((_BC, _NS, _C), lambda i: (i, 0, 0)),
            pl.BlockSpec((_C, 64), lambda i: (0, 0)),
            pl.BlockSpec((1, 64), lambda i: (0, 0)),
            pl.BlockSpec((64, 128), lambda i: (0, 0)),
            pl.BlockSpec((1, 128), lambda i: (0, 0)),
            pl.BlockSpec((128, 256), lambda i: (0, 0)),
            pl.BlockSpec((1, 256), lambda i: (0, 0)),
        ],
        out_specs=pl.BlockSpec((_BC, 256), lambda i: (i, 0)),
        out_shape=jax.ShapeDtypeStruct((_B * _NPT, 256), jnp.float32),
    )(g3, W1, b1.reshape(1, 64), W2, b2.reshape(1, 128), W3, b3.reshape(1, 256))
    return out.reshape(_B, _NPT, 256)


def kernel(xyz, features, W1, b1, W2, b2, W3, b3):
    cent = _run_fps(xyz)
    c3 = cent.reshape(_NPT, 4, 4)[:, :, :3]
    new_xyz = jnp.transpose(c3, (1, 0, 2))

    nx_flat = jnp.transpose(c3, (1, 2, 0)).reshape(-1)
    xyzt_flat = jnp.transpose(xyz, (0, 2, 1)).reshape(-1)
    feat2d = features.reshape(_B * _N, _C)
    grouped = _make_bq()(nx_flat, xyzt_flat, feat2d)

    new_features = _run_mlp(grouped, W1, b1, W2, b2, W3, b3)
    return (new_xyz, new_features)


# speedup vs baseline: 1.0604x; 1.0604x over previous
import functools

import jax
import jax.numpy as jnp
import numpy as np
from jax import lax
from jax.experimental import pallas as pl
from jax.experimental.pallas import tpu as pltpu
from jax.experimental.pallas import tpu_sc as plsc

_B = 4
_N = 8192
_C = 64
_NPT = 1024
_NS = 32
_R2 = np.float32(0.2 * 0.2)


def _redmax(a):
    for w in (512, 256, 128):
        a = jnp.maximum(a[:, :, :w], a[:, :, w:])
    a32 = jnp.max(a.reshape(_B * 8, 128), axis=1, keepdims=True)
    return jnp.max(a32.reshape(_B, 8, 1), axis=1, keepdims=True)


def _redmin(a):
    for w in (512, 256, 128):
        a = jnp.minimum(a[:, :, :w], a[:, :, w:])
    a32 = jnp.min(a.reshape(_B * 8, 128), axis=1, keepdims=True)
    return jnp.min(a32.reshape(_B, 8, 1), axis=1, keepdims=True)


def _fps_body(xs_ref, ys_ref, zs_ref, cent_ref, dist_ref):
    flat3 = (lax.broadcasted_iota(jnp.int32, (_B, 8, 1024), 1) * 1024
             + lax.broadcasted_iota(jnp.int32, (_B, 8, 1024), 2))
    lane16 = lax.broadcasted_iota(jnp.int32, (1, 16), 1)
    dist_ref[...] = jnp.full((_B, 8, 1024), 1e10, jnp.float32)

    def step(i, carry):
        cx4, cy4, cz4 = carry
        crow = jnp.zeros((1, 16), jnp.float32)
        for b in range(_B):
            crow = jnp.where(lane16 == (4 * b + 0), cx4[b, 0:1, :], crow)
            crow = jnp.where(lane16 == (4 * b + 1), cy4[b, 0:1, :], crow)
            crow = jnp.where(lane16 == (4 * b + 2), cz4[b, 0:1, :], crow)
        cent_ref[pl.ds(i, 1), :] = crow
        xv = xs_ref[...]
        yv = ys_ref[...]
        zv = zs_ref[...]
        dx = xv - cx4
        dy = yv - cy4
        dz = zv - cz4
        d = (dx * dx + dy * dy) + dz * dz
        d4n = jnp.minimum(dist_ref[...], d)
        dist_ref[...] = d4n
        m4 = _redmax(d4n)
        cand = jnp.where(d4n == m4, flat3, jnp.int32(_N))
        fmin4 = _redmin(cand)
        maskf = flat3 == fmin4
        ncx = _redmax(jnp.where(maskf, xv, -1.0))
        ncy = _redmax(jnp.where(maskf, yv, -1.0))
        ncz = _redmax(jnp.where(maskf, zv, -1.0))
        return (ncx, ncy, ncz)

    c0 = tuple(r[:, 0:1, 0:1] for r in (xs_ref, ys_ref, zs_ref))
    lax.fori_loop(0, _NPT, step, c0)


@functools.partial(jax.jit)
def _run_fps(xyz):
    planes = jnp.transpose(xyz, (0, 2, 1)).reshape(_B, 3, 8, 1024)
    cent = pl.pallas_call(
        _fps_body,
        out_shape=jax.ShapeDtypeStruct((_NPT, 16), jnp.float32),
        scratch_shapes=[pltpu.VMEM((_B, 8, 1024), jnp.float32)],
    )(planes[:, 0], planes[:, 1], planes[:, 2])
    return cent


_NW = 32


def _bq_body(nx_hbm, xyzt_hbm, feat_hbm, out_hbm,
             cx_v, cy_v, cz_v, xs_v, ys_v, zs_v, idx_v, gbuf_v, sem):
    cid = lax.axis_index("c")
    sid = lax.axis_index("s")
    wid = sid * 2 + cid
    b = wid // 8
    part = wid % 8
    lane = lax.iota(jnp.int32, 16)

    pltpu.sync_copy(nx_hbm.at[pl.ds((b * 3 + 0) * _NPT + part * 128, 128)], cx_v)
    pltpu.sync_copy(nx_hbm.at[pl.ds((b * 3 + 1) * _NPT + part * 128, 128)], cy_v)
    pltpu.sync_copy(nx_hbm.at[pl.ds((b * 3 + 2) * _NPT + part * 128, 128)], cz_v)
    pltpu.sync_copy(xyzt_hbm.at[pl.ds((b * 3 + 0) * _N, _N)], xs_v)
    pltpu.sync_copy(xyzt_hbm.at[pl.ds((b * 3 + 1) * _N, _N)], ys_v)
    pltpu.sync_copy(xyzt_hbm.at[pl.ds((b * 3 + 2) * _N, _N)], zs_v)

    rowbase = b * _N
    zeros16 = jnp.zeros((16,), jnp.int32)

    def one_centroid(si, _):
        g = si // 16
        j = si - g * 16
        js = jnp.full((16,), j, jnp.int32)
        cxs = cx_v[pl.ds(g * 16, 16)].at[js].get(mode="promise_in_bounds")
        cys = cy_v[pl.ds(g * 16, 16)].at[js].get(mode="promise_in_bounds")
        czs = cz_v[pl.ds(g * 16, 16)].at[js].get(mode="promise_in_bounds")

        def chunk(k, cnt):
            for u in range(4):
                base = k * 16 + u * 16
                dx = xs_v[pl.ds(base, 16)] - cxs
                dy = ys_v[pl.ds(base, 16)] - cys
                dz = zs_v[pl.ds(base, 16)] - czs
                d = (dx * dx + dy * dy) + dz * dz
                inr = d <= _R2
                npts = plsc.all_reduce_population_count(inr)
                ivec = lane + jnp.full((16,), base + rowbase, jnp.int32)
                off = si * _NS + jnp.minimum(cnt, _NS)
                plsc.store_compressed(idx_v.at[pl.ds(off, 16)], ivec, mask=inr)
                cnt = cnt + npts[0]
            return cnt

        cnt = jnp.int32(0)
        k_cur = jnp.int32(0)
        for rl in (16, 16, 16, 16, 16, 16, 32, 64, 128, 192):
            k_end = jnp.where(cnt >= _NS, k_cur, k_cur + rl)
            cnt = pl.loop(k_cur, k_end, init_carry=cnt, step=4)(chunk)
            k_cur = k_end

        cnt32 = jnp.minimum(cnt, _NS)
        v0 = idx_v[pl.ds(si * _NS, 16)]
        first_raw = v0.at[zeros16].get(mode="promise_in_bounds")
        firsts = jnp.where(cnt32 > 0, first_raw, jnp.full((16,), rowbase, jnp.int32))
        for h in range(2):
            cur = idx_v[pl.ds(si * _NS + h * 16, 16)]
            gl = lane + h * 16
            idx_v[pl.ds(si * _NS + h * 16, 16)] = jnp.where(gl < cnt32, cur, firsts)
        return 0

    lax.fori_loop(0, 128, one_centroid, 0)

    out_base = (b * _NPT + part * 128) * _NS

    def gather_chunk(ci, _):
        pltpu.async_copy(feat_hbm.at[idx_v.at[pl.ds(ci * 128, 128)]], gbuf_v, sem).wait()
        pltpu.sync_copy(gbuf_v, out_hbm.at[pl.ds(out_base + ci * 128, 128)])
        return 0

    lax.fori_loop(0, _NS, gather_chunk, 0)


def _make_bq():
    mesh = plsc.VectorSubcoreMesh(core_axis_name="c", subcore_axis_name="s")
    return functools.partial(
        pl.kernel,
        mesh=mesh,
        compiler_params=pltpu.CompilerParams(needs_layout_passes=False,
                                             use_tc_tiling_on_sc=False),
        out_type=jax.ShapeDtypeStruct((_B * _NPT * _NS, _C), jnp.float32),
        scratch_types=[
            pltpu.VMEM((128,), jnp.float32),
            pltpu.VMEM((128,), jnp.float32),
            pltpu.VMEM((128,), jnp.float32),
            pltpu.VMEM((_N,), jnp.float32),
            pltpu.VMEM((_N,), jnp.float32),
            pltpu.VMEM((_N,), jnp.float32),
            pltpu.VMEM((128 * _NS + 16,), jnp.int32),
            pltpu.VMEM((128, _C), jnp.float32),
            pltpu.SemaphoreType.DMA,
        ],
    )(_bq_body)


_BC = 64


def _mlp_body(g_ref, w1_ref, b1_ref, w2_ref, b2_ref, w3_ref, b3_ref, out_ref):
    x = g_ref[...].reshape(_BC * _NS, _C)
    h = jnp.dot(x, w1_ref[...], preferred_element_type=jnp.float32) + b1_ref[...]
    h = jnp.maximum(h, 0.0)
    h = jnp.dot(h, w2_ref[...], preferred_element_type=jnp.float32) + b2_ref[...]
    h = jnp.maximum(h, 0.0)
    h = jnp.dot(h, w3_ref[...], preferred_element_type=jnp.float32) + b3_ref[...]
    out_ref[...] = jnp.max(h.reshape(_BC, _NS, 256), axis=1)


def _run_mlp(grouped, W1, b1, W2, b2, W3, b3):
    g3 = grouped.reshape(_B * _NPT, _NS, _C)
    out = pl.pallas_call(
        _mlp_body,
        grid=(_B * _NPT // _BC,),
        in_specs=[
            pl.BlockSpec((_BC, _NS, _C), lambda i: (i, 0, 0)),
            pl.BlockSpec((_C, 64), lambda i: (0, 0)),
            pl.BlockSpec((1, 64), lambda i: (0, 0)),
            pl.BlockSpec((64, 128), lambda i: (0, 0)),
            pl.BlockSpec((1, 128), lambda i: (0, 0)),
            pl.BlockSpec((128, 256), lambda i: (0, 0)),
            pl.BlockSpec((1, 256), lambda i: (0, 0)),
        ],
        out_specs=pl.BlockSpec((_BC, 256), lambda i: (i, 0)),
        out_shape=jax.ShapeDtypeStruct((_B * _NPT, 256), jnp.float32),
    )(g3, W1, b1.reshape(1, 64), W2, b2.reshape(1, 128), W3, b3.reshape(1, 256))
    return out.reshape(_B, _NPT, 256)


def kernel(xyz, features, W1, b1, W2, b2, W3, b3):
    cent = _run_fps(xyz)
    c3 = cent.reshape(_NPT, 4, 4)[:, :, :3]
    new_xyz = jnp.transpose(c3, (1, 0, 2))

    nx_flat = jnp.transpose(c3, (1, 2, 0)).reshape(-1)
    xyzt_flat = jnp.transpose(xyz, (0, 2, 1)).reshape(-1)
    feat2d = features.reshape(_B * _N, _C)
    grouped = _make_bq()(nx_flat, xyzt_flat, feat2d)

    new_features = _run_mlp(grouped, W1, b1, W2, b2, W3, b3)
    return (new_xyz, new_features)


# speedup vs baseline: 1.2784x; 1.2056x over previous
import functools

import jax
import jax.numpy as jnp
import numpy as np
from jax import lax
from jax.experimental import pallas as pl
from jax.experimental.pallas import tpu as pltpu
from jax.experimental.pallas import tpu_sc as plsc

_B = 4
_N = 8192
_C = 64
_NPT = 1024
_NS = 32
_R2 = np.float32(0.2 * 0.2)


def _redmax(a):
    for w in (512, 256, 128):
        a = jnp.maximum(a[:, :, :w], a[:, :, w:])
    a32 = jnp.max(a.reshape(_B * 8, 128), axis=1, keepdims=True)
    return jnp.max(a32.reshape(_B, 8, 1), axis=1, keepdims=True)


def _redmin(a):
    for w in (512, 256, 128):
        a = jnp.minimum(a[:, :, :w], a[:, :, w:])
    a32 = jnp.min(a.reshape(_B * 8, 128), axis=1, keepdims=True)
    return jnp.min(a32.reshape(_B, 8, 1), axis=1, keepdims=True)


def _fps_body(xs_ref, ys_ref, zs_ref, cent_ref, dist_ref):
    flat3 = (lax.broadcasted_iota(jnp.int32, (_B, 8, 1024), 1) * 1024
             + lax.broadcasted_iota(jnp.int32, (_B, 8, 1024), 2))
    lane16 = lax.broadcasted_iota(jnp.int32, (1, 16), 1)
    dist_ref[...] = jnp.full((_B, 8, 1024), 1e10, jnp.float32)

    def step(i, carry):
        cx4, cy4, cz4 = carry
        crow = jnp.zeros((1, 16), jnp.float32)
        for b in range(_B):
            crow = jnp.where(lane16 == (4 * b + 0), cx4[b, 0:1, :], crow)
            crow = jnp.where(lane16 == (4 * b + 1), cy4[b, 0:1, :], crow)
            crow = jnp.where(lane16 == (4 * b + 2), cz4[b, 0:1, :], crow)
        cent_ref[pl.ds(i, 1), :] = crow
        xv = xs_ref[...]
        yv = ys_ref[...]
        zv = zs_ref[...]
        dx = xv - cx4
        dy = yv - cy4
        dz = zv - cz4
        d = (dx * dx + dy * dy) + dz * dz
        d4n = jnp.minimum(dist_ref[...], d)
        dist_ref[...] = d4n
        m4 = _redmax(d4n)
        cand = jnp.where(d4n == m4, flat3, jnp.int32(_N))
        fmin4 = _redmin(cand)
        maskf = flat3 == fmin4
        ncx = _redmax(jnp.where(maskf, xv, -1.0))
        ncy = _redmax(jnp.where(maskf, yv, -1.0))
        ncz = _redmax(jnp.where(maskf, zv, -1.0))
        return (ncx, ncy, ncz)

    c0 = tuple(r[:, 0:1, 0:1] for r in (xs_ref, ys_ref, zs_ref))
    lax.fori_loop(0, _NPT, step, c0)


@functools.partial(jax.jit)
def _run_fps(xyz):
    planes = jnp.transpose(xyz, (0, 2, 1)).reshape(_B, 3, 8, 1024)
    cent = pl.pallas_call(
        _fps_body,
        out_shape=jax.ShapeDtypeStruct((_NPT, 16), jnp.float32),
        scratch_shapes=[pltpu.VMEM((_B, 8, 1024), jnp.float32)],
    )(planes[:, 0], planes[:, 1], planes[:, 2])
    return cent


_NW = 32


def _bq_body(nx_hbm, xyzt_hbm, feat_hbm, out_hbm,
             cx_v, cy_v, cz_v, xs_v, ys_v, zs_v, idx_v, gbuf_v, sem):
    cid = lax.axis_index("c")
    sid = lax.axis_index("s")
    wid = sid * 2 + cid
    b = wid // 8
    part = wid % 8
    lane = lax.iota(jnp.int32, 16)

    pltpu.sync_copy(nx_hbm.at[pl.ds((b * 3 + 0) * _NPT + part * 128, 128)], cx_v)
    pltpu.sync_copy(nx_hbm.at[pl.ds((b * 3 + 1) * _NPT + part * 128, 128)], cy_v)
    pltpu.sync_copy(nx_hbm.at[pl.ds((b * 3 + 2) * _NPT + part * 128, 128)], cz_v)
    pltpu.sync_copy(xyzt_hbm.at[pl.ds((b * 3 + 0) * _N, _N)], xs_v)
    pltpu.sync_copy(xyzt_hbm.at[pl.ds((b * 3 + 1) * _N, _N)], ys_v)
    pltpu.sync_copy(xyzt_hbm.at[pl.ds((b * 3 + 2) * _N, _N)], zs_v)

    rowbase = b * _N
    zeros16 = jnp.zeros((16,), jnp.int32)

    def one_centroid(si, _):
        g = si // 16
        j = si - g * 16
        js = jnp.full((16,), j, jnp.int32)
        cxs = cx_v[pl.ds(g * 16, 16)].at[js].get(mode="promise_in_bounds")
        cys = cy_v[pl.ds(g * 16, 16)].at[js].get(mode="promise_in_bounds")
        czs = cz_v[pl.ds(g * 16, 16)].at[js].get(mode="promise_in_bounds")

        def chunk(k, cnt):
            base = k * 16
            dx = xs_v[pl.ds(base, 16)] - cxs
            dy = ys_v[pl.ds(base, 16)] - cys
            dz = zs_v[pl.ds(base, 16)] - czs
            d = (dx * dx + dy * dy) + dz * dz
            inr = d <= _R2
            npts = plsc.all_reduce_population_count(inr)
            ivec = lane + jnp.full((16,), base + rowbase, jnp.int32)
            off = si * _NS + jnp.minimum(cnt, _NS)
            plsc.store_compressed(idx_v.at[pl.ds(off, 16)], ivec, mask=inr)
            return cnt + npts[0]

        cnt = jnp.int32(0)
        k_cur = jnp.int32(0)
        for rl in (16, 16, 16, 16, 16, 16, 32, 64, 128, 192):
            k_end = jnp.where(cnt >= _NS, k_cur, k_cur + rl)
            cnt = pl.loop(k_cur, k_end, init_carry=cnt)(chunk)
            k_cur = k_end

        cnt32 = jnp.minimum(cnt, _NS)
        v0 = idx_v[pl.ds(si * _NS, 16)]
        first_raw = v0.at[zeros16].get(mode="promise_in_bounds")
        firsts = jnp.where(cnt32 > 0, first_raw, jnp.full((16,), rowbase, jnp.int32))
        for h in range(2):
            cur = idx_v[pl.ds(si * _NS + h * 16, 16)]
            gl = lane + h * 16
            idx_v[pl.ds(si * _NS + h * 16, 16)] = jnp.where(gl < cnt32, cur, firsts)
        return 0

    lax.fori_loop(0, 128, one_centroid, 0)

    out_base = (b * _NPT + part * 128) * _NS

    def gather_chunk(ci, _):
        pltpu.async_copy(feat_hbm.at[idx_v.at[pl.ds(ci * 128, 128)]], gbuf_v, sem).wait()
        pltpu.sync_copy(gbuf_v, out_hbm.at[pl.ds(out_base + ci * 128, 128)])
        return 0

    lax.fori_loop(0, _NS, gather_chunk, 0)


def _make_bq():
    mesh = plsc.VectorSubcoreMesh(core_axis_name="c", subcore_axis_name="s")
    return functools.partial(
        pl.kernel,
        mesh=mesh,
        compiler_params=pltpu.CompilerParams(needs_layout_passes=False,
                                             use_tc_tiling_on_sc=False),
        out_type=jax.ShapeDtypeStruct((_B * _NPT * _NS, _C), jnp.float32),
        scratch_types=[
            pltpu.VMEM((128,), jnp.float32),
            pltpu.VMEM((128,), jnp.float32),
            pltpu.VMEM((128,), jnp.float32),
            pltpu.VMEM((_N,), jnp.float32),
            pltpu.VMEM((_N,), jnp.float32),
            pltpu.VMEM((_N,), jnp.float32),
            pltpu.VMEM((128 * _NS + 16,), jnp.int32),
            pltpu.VMEM((128, _C), jnp.float32),
            pltpu.SemaphoreType.DMA,
        ],
    )(_bq_body)


_BC = 128


def _mlp_body(g_ref, w1_ref, b1_ref, w2_ref, b2_ref, w3_ref, b3_ref, out_ref):
    x = g_ref[...].reshape(_BC * _NS, _C)
    h = jnp.dot(x, w1_ref[...], preferred_element_type=jnp.float32) + b1_ref[...]
    h = jnp.maximum(h, 0.0)
    h = jnp.dot(h, w2_ref[...], preferred_element_type=jnp.float32) + b2_ref[...]
    h = jnp.maximum(h, 0.0)
    h = jnp.dot(h, w3_ref[...], preferred_element_type=jnp.float32) + b3_ref[...]
    out_ref[...] = jnp.max(h.reshape(_BC, _NS, 256), axis=1)


def _run_mlp(grouped, W1, b1, W2, b2, W3, b3):
    g3 = grouped.reshape(_B * _NPT, _NS, _C)
    out = pl.pallas_call(
        _mlp_body,
        grid=(_B * _NPT // _BC,),
        in_specs=[
            pl.BlockSpec((_BC, _NS, _C), lambda i: (i, 0, 0)),
            pl.BlockSpec((_C, 64), lambda i: (0, 0)),
            pl.BlockSpec((1, 64), lambda i: (0, 0)),
            pl.BlockSpec((64, 128), lambda i: (0, 0)),
            pl.BlockSpec((1, 128), lambda i: (0, 0)),
            pl.BlockSpec((128, 256), lambda i: (0, 0)),
            pl.BlockSpec((1, 256), lambda i: (0, 0)),
        ],
        out_specs=pl.BlockSpec((_BC, 256), lambda i: (i, 0)),
        out_shape=jax.ShapeDtypeStruct((_B * _NPT, 256), jnp.float32),
    )(g3, W1, b1.reshape(1, 64), W2, b2.reshape(1, 128), W3, b3.reshape(1, 256))
    return out.reshape(_B, _NPT, 256)


def kernel(xyz, features, W1, b1, W2, b2, W3, b3):
    cent = _run_fps(xyz)
    c3 = cent.reshape(_NPT, 4, 4)[:, :, :3]
    new_xyz = jnp.transpose(c3, (1, 0, 2))

    nx_flat = jnp.transpose(c3, (1, 2, 0)).reshape(-1)
    xyzt_flat = jnp.transpose(xyz, (0, 2, 1)).reshape(-1)
    feat2d = features.reshape(_B * _N, _C)
    grouped = _make_bq()(nx_flat, xyzt_flat, feat2d)

    new_features = _run_mlp(grouped, W1, b1, W2, b2, W3, b3)
    return (new_xyz, new_features)


# speedup vs baseline: 1.3219x; 1.0340x over previous
import functools

import jax
import jax.numpy as jnp
import numpy as np
from jax import lax
from jax.experimental import pallas as pl
from jax.experimental.pallas import tpu as pltpu
from jax.experimental.pallas import tpu_sc as plsc

_B = 4
_N = 8192
_C = 64
_NPT = 1024
_NS = 32
_R2 = np.float32(0.2 * 0.2)


def _redmax(a):
    for w in (512, 256, 128):
        a = jnp.maximum(a[:, :, :w], a[:, :, w:])
    a32 = jnp.max(a.reshape(_B * 8, 128), axis=1, keepdims=True)
    return jnp.max(a32.reshape(_B, 8, 1), axis=1, keepdims=True)


def _redmin(a):
    for w in (512, 256, 128):
        a = jnp.minimum(a[:, :, :w], a[:, :, w:])
    a32 = jnp.min(a.reshape(_B * 8, 128), axis=1, keepdims=True)
    return jnp.min(a32.reshape(_B, 8, 1), axis=1, keepdims=True)


def _fps_body(xs_ref, ys_ref, zs_ref, cent_ref, dist_ref):
    flat3 = (lax.broadcasted_iota(jnp.int32, (_B, 8, 1024), 1) * 1024
             + lax.broadcasted_iota(jnp.int32, (_B, 8, 1024), 2))
    lane16 = lax.broadcasted_iota(jnp.int32, (1, 16), 1)
    dist_ref[...] = jnp.full((_B, 8, 1024), 1e10, jnp.float32)

    def step(i, carry):
        cx4, cy4, cz4 = carry
        crow = jnp.zeros((1, 16), jnp.float32)
        for b in range(_B):
            crow = jnp.where(lane16 == (4 * b + 0), cx4[b, 0:1, :], crow)
            crow = jnp.where(lane16 == (4 * b + 1), cy4[b, 0:1, :], crow)
            crow = jnp.where(lane16 == (4 * b + 2), cz4[b, 0:1, :], crow)
        cent_ref[pl.ds(i, 1), :] = crow
        xv = xs_ref[...]
        yv = ys_ref[...]
        zv = zs_ref[...]
        dx = xv - cx4
        dy = yv - cy4
        dz = zv - cz4
        d = (dx * dx + dy * dy) + dz * dz
        d4n = jnp.minimum(dist_ref[...], d)
        dist_ref[...] = d4n
        m4 = _redmax(d4n)
        cand = jnp.where(d4n == m4, flat3, jnp.int32(_N))
        fmin4 = _redmin(cand)
        maskf = flat3 == fmin4
        ncx = _redmax(jnp.where(maskf, xv, -1.0))
        ncy = _redmax(jnp.where(maskf, yv, -1.0))
        ncz = _redmax(jnp.where(maskf, zv, -1.0))
        return (ncx, ncy, ncz)

    c0 = tuple(r[:, 0:1, 0:1] for r in (xs_ref, ys_ref, zs_ref))
    lax.fori_loop(0, _NPT, step, c0)


@functools.partial(jax.jit)
def _run_fps(xyz):
    planes = jnp.transpose(xyz, (0, 2, 1)).reshape(_B, 3, 8, 1024)
    cent = pl.pallas_call(
        _fps_body,
        out_shape=jax.ShapeDtypeStruct((_NPT, 16), jnp.float32),
        scratch_shapes=[pltpu.VMEM((_B, 8, 1024), jnp.float32)],
    )(planes[:, 0], planes[:, 1], planes[:, 2])
    return cent


_NW = 32


def _bq_body(nx_hbm, xyzt_hbm, feat_hbm, out_hbm,
             cx_v, cy_v, cz_v, xs_v, ys_v, zs_v, idx_v, gbuf_v, sem):
    cid = lax.axis_index("c")
    sid = lax.axis_index("s")
    wid = sid * 2 + cid
    b = wid // 8
    part = wid % 8
    lane = lax.iota(jnp.int32, 16)

    pltpu.sync_copy(nx_hbm.at[pl.ds((b * 3 + 0) * _NPT, _NPT)], cx_v)
    pltpu.sync_copy(nx_hbm.at[pl.ds((b * 3 + 1) * _NPT, _NPT)], cy_v)
    pltpu.sync_copy(nx_hbm.at[pl.ds((b * 3 + 2) * _NPT, _NPT)], cz_v)
    pltpu.sync_copy(xyzt_hbm.at[pl.ds((b * 3 + 0) * _N, _N)], xs_v)
    pltpu.sync_copy(xyzt_hbm.at[pl.ds((b * 3 + 1) * _N, _N)], ys_v)
    pltpu.sync_copy(xyzt_hbm.at[pl.ds((b * 3 + 2) * _N, _N)], zs_v)

    rowbase = b * _N
    zeros16 = jnp.zeros((16,), jnp.int32)

    def one_centroid(si, _):
        sg = part + 8 * si
        g = sg // 16
        j = sg - g * 16
        js = jnp.full((16,), j, jnp.int32)
        cxs = cx_v[pl.ds(g * 16, 16)].at[js].get(mode="promise_in_bounds")
        cys = cy_v[pl.ds(g * 16, 16)].at[js].get(mode="promise_in_bounds")
        czs = cz_v[pl.ds(g * 16, 16)].at[js].get(mode="promise_in_bounds")

        def chunk(k, cnt):
            base = k * 16
            dx = xs_v[pl.ds(base, 16)] - cxs
            dy = ys_v[pl.ds(base, 16)] - cys
            dz = zs_v[pl.ds(base, 16)] - czs
            d = (dx * dx + dy * dy) + dz * dz
            inr = d <= _R2
            npts = plsc.all_reduce_population_count(inr)
            ivec = lane + jnp.full((16,), base + rowbase, jnp.int32)
            off = si * _NS + jnp.minimum(cnt, _NS)
            plsc.store_compressed(idx_v.at[pl.ds(off, 16)], ivec, mask=inr)
            return cnt + npts[0]

        cnt = jnp.int32(0)
        k_cur = jnp.int32(0)
        for rl in (16, 16, 16, 16, 16, 16, 32, 64, 128, 192):
            k_end = jnp.where(cnt >= _NS, k_cur, k_cur + rl)
            cnt = pl.loop(k_cur, k_end, init_carry=cnt)(chunk)
            k_cur = k_end

        cnt32 = jnp.minimum(cnt, _NS)
        v0 = idx_v[pl.ds(si * _NS, 16)]
        first_raw = v0.at[zeros16].get(mode="promise_in_bounds")
        firsts = jnp.where(cnt32 > 0, first_raw, jnp.full((16,), rowbase, jnp.int32))
        for h in range(2):
            cur = idx_v[pl.ds(si * _NS + h * 16, 16)]
            gl = lane + h * 16
            idx_v[pl.ds(si * _NS + h * 16, 16)] = jnp.where(gl < cnt32, cur, firsts)
        return 0

    lax.fori_loop(0, 128, one_centroid, 0)

    def gather_chunk(ci, _):
        pltpu.async_copy(feat_hbm.at[idx_v.at[pl.ds(ci * 128, 128)]], gbuf_v, sem).wait()
        for u in range(4):
            sg = part + 8 * (ci * 4 + u)
            pltpu.sync_copy(gbuf_v.at[pl.ds(u * _NS, _NS)],
                            out_hbm.at[pl.ds((b * _NPT + sg) * _NS, _NS)])
        return 0

    lax.fori_loop(0, _NS, gather_chunk, 0)


def _make_bq():
    mesh = plsc.VectorSubcoreMesh(core_axis_name="c", subcore_axis_name="s")
    return functools.partial(
        pl.kernel,
        mesh=mesh,
        compiler_params=pltpu.CompilerParams(needs_layout_passes=False,
                                             use_tc_tiling_on_sc=False),
        out_type=jax.ShapeDtypeStruct((_B * _NPT * _NS, _C), jnp.float32),
        scratch_types=[
            pltpu.VMEM((_NPT,), jnp.float32),
            pltpu.VMEM((_NPT,), jnp.float32),
            pltpu.VMEM((_NPT,), jnp.float32),
            pltpu.VMEM((_N,), jnp.float32),
            pltpu.VMEM((_N,), jnp.float32),
            pltpu.VMEM((_N,), jnp.float32),
            pltpu.VMEM((128 * _NS + 16,), jnp.int32),
            pltpu.VMEM((128, _C), jnp.float32),
            pltpu.SemaphoreType.DMA,
        ],
    )(_bq_body)


_BC = 128


def _mlp_body(g_ref, w1_ref, b1_ref, w2_ref, b2_ref, w3_ref, b3_ref, out_ref):
    x = g_ref[...].reshape(_BC * _NS, _C)
    h = jnp.dot(x, w1_ref[...], preferred_element_type=jnp.float32) + b1_ref[...]
    h = jnp.maximum(h, 0.0)
    h = jnp.dot(h, w2_ref[...], preferred_element_type=jnp.float32) + b2_ref[...]
    h = jnp.maximum(h, 0.0)
    h = jnp.dot(h, w3_ref[...], preferred_element_type=jnp.float32) + b3_ref[...]
    out_ref[...] = jnp.max(h.reshape(_BC, _NS, 256), axis=1)


def _run_mlp(grouped, W1, b1, W2, b2, W3, b3):
    g3 = grouped.reshape(_B * _NPT, _NS, _C)
    out = pl.pallas_call(
        _mlp_body,
        grid=(_B * _NPT // _BC,),
        in_specs=[
            pl.BlockSpec((_BC, _NS, _C), lambda i: (i, 0, 0)),
            pl.BlockSpec((_C, 64), lambda i: (0, 0)),
            pl.BlockSpec((1, 64), lambda i: (0, 0)),
            pl.BlockSpec((64, 128), lambda i: (0, 0)),
            pl.BlockSpec((1, 128), lambda i: (0, 0)),
            pl.BlockSpec((128, 256), lambda i: (0, 0)),
            pl.BlockSpec((1, 256), lambda i: (0, 0)),
        ],
        out_specs=pl.BlockSpec((_BC, 256), lambda i: (i, 0)),
        out_shape=jax.ShapeDtypeStruct((_B * _NPT, 256), jnp.float32),
    )(g3, W1, b1.reshape(1, 64), W2, b2.reshape(1, 128), W3, b3.reshape(1, 256))
    return out.reshape(_B, _NPT, 256)


def kernel(xyz, features, W1, b1, W2, b2, W3, b3):
    cent = _run_fps(xyz)
    c3 = cent.reshape(_NPT, 4, 4)[:, :, :3]
    new_xyz = jnp.transpose(c3, (1, 0, 2))

    nx_flat = jnp.transpose(c3, (1, 2, 0)).reshape(-1)
    xyzt_flat = jnp.transpose(xyz, (0, 2, 1)).reshape(-1)
    feat2d = features.reshape(_B * _N, _C)
    grouped = _make_bq()(nx_flat, xyzt_flat, feat2d)

    new_features = _run_mlp(grouped, W1, b1, W2, b2, W3, b3)
    return (new_xyz, new_features)
